# baseline (speedup 1.0000x reference)
import functools

import jax
import jax.numpy as jnp
from jax import lax
from jax.experimental import pallas as pl
from jax.experimental.pallas import tpu as pltpu
from jax.experimental.pallas import tpu_sc as plsc

_NC = 2
_NS = 16
_NW = _NC * _NS
_LN_EPS = 1e-5


def _layer_norm_2d(x, g, b):
    m = x.mean(axis=-1, keepdims=True)
    v = ((x - m) ** 2).mean(axis=-1, keepdims=True)
    return (x - m) * lax.rsqrt(v + _LN_EPS) * g + b


def _proj_body(S, turn_t, slot_t, value_t, wt, wbs, wsv, bt, bsv, bbs,
               edge_t, etypes, turn_pre, cat_table, edge_attr):
    f32 = jnp.float32
    ns = slot_t.shape[0]
    turn_pre[...] = jnp.dot(turn_t[...], wt[...], preferred_element_type=f32) + bt[...]
    s_sv = jnp.dot(slot_t[...], wsv[...], preferred_element_type=f32)
    s_bs = jnp.dot(slot_t[...], wbs[...], preferred_element_type=f32) * (1.0 / S)
    cat_table[:ns, :] = jnp.concatenate([s_sv, s_bs], axis=1)
    v_sv = jnp.dot(value_t[...], wsv[...], preferred_element_type=f32) + bsv[...]
    v_bs = (jnp.dot(value_t[...], wbs[...], preferred_element_type=f32) + bbs[...]) * (1.0 / S)
    cat_table[ns:, :] = jnp.concatenate([v_sv, v_bs], axis=1)
    ne = edge_t.shape[0]
    oh = (etypes[...] == lax.broadcasted_iota(jnp.int32, (etypes.shape[0], ne), 1)).astype(f32)
    edge_attr[...] = jnp.dot(oh, edge_t[...], preferred_element_type=f32)


def _project(S, turn_table, slot_table, value_table, W_turn, W_bs, W_sv,
             b_turn, b_sv, b_bs, edge_table, etypes_col):
    B = etypes_col.shape[0]
    H = turn_table.shape[1]
    EA = edge_table.shape[1]
    nrows = slot_table.shape[0] + value_table.shape[0]
    out_shapes = [
        jax.ShapeDtypeStruct((turn_table.shape[0], H), jnp.float32),
        jax.ShapeDtypeStruct((nrows, 2 * H), jnp.float32),
        jax.ShapeDtypeStruct((B, EA), jnp.float32),
    ]
    return pl.pallas_call(
        functools.partial(_proj_body, S),
        out_shape=out_shapes,
    )(turn_table, slot_table, value_table, W_turn, W_bs, W_sv,
      b_turn, b_sv, b_bs, edge_table, etypes_col)


def _sc_gather(cat_ids, cat_table, S, H):
    B, NI = cat_ids.shape
    b_per_w = B // _NW
    IC = 8
    HC = H // 16

    mesh = plsc.VectorSubcoreMesh(core_axis_name="c", subcore_axis_name="s", num_cores=_NC, num_subcores=_NS)

    @functools.partial(
        pl.kernel,
        mesh=mesh,
        out_type=jax.ShapeDtypeStruct((B, S + 1, H), jnp.float32),
        scratch_types=[
            pltpu.VMEM((IC, NI), jnp.int32),
            pltpu.VMEM((NI, 2 * H), jnp.float32),
            pltpu.VMEM((S + 1, H), jnp.float32),
            pltpu.SemaphoreType.DMA,
        ],
    )
    def k(ids_hbm, cat_hbm, out_hbm, ids_v, rows_v, outb_v, sem):
        wid = lax.axis_index("s") * _NC + lax.axis_index("c")
        base = wid * b_per_w
        zero = jnp.zeros((16,), jnp.float32)

        def body_chunk(c, carry):
            b0 = base + c * IC
            pltpu.sync_copy(ids_hbm.at[pl.ds(b0, IC)], ids_v)

            def body_b(j, carry2):
                cp = pltpu.async_copy(cat_hbm.at[ids_v.at[j]], rows_v, sem)
                cp.wait()
                for h in range(HC):
                    outb_v[0, pl.ds(h * 16, 16)] = zero

                def body_s(s, c2):
                    for h in range(HC):
                        sv = (rows_v[s, pl.ds(h * 16, 16)]
                              + rows_v[S + s, pl.ds(h * 16, 16)])
                        outb_v[s + 1, pl.ds(h * 16, 16)] = sv
                        bs = (rows_v[s, pl.ds(H + h * 16, 16)]
                              + rows_v[S + s, pl.ds(H + h * 16, 16)])
                        acc = outb_v[0, pl.ds(h * 16, 16)]
                        outb_v[0, pl.ds(h * 16, 16)] = acc + bs
                    return c2

                lax.fori_loop(0, S, body_s, 0)
                pltpu.sync_copy(outb_v, out_hbm.at[b0 + j])
                return carry2

            lax.fori_loop(0, IC, body_b, 0)
            return carry

        lax.fori_loop(0, b_per_w // IC, body_chunk, 0)

    return k(cat_ids, cat_table)


def _ln_body(p_ref, tid_ref, tpre_ref, g_ref, b_ref, out_ref):
    bsz, sp1, H = p_ref.shape
    nt = tpre_ref.shape[0]
    g = g_ref[...]
    b = b_ref[...]
    x = p_ref[...].reshape(bsz * sp1, H)
    out_ref[:, 1:, :] = _layer_norm_2d(x, g, b).reshape(bsz, sp1, H)
    oh = (tid_ref[...] == lax.broadcasted_iota(jnp.int32, (bsz, nt), 1)).astype(jnp.float32)
    t = jnp.dot(oh, tpre_ref[...], preferred_element_type=jnp.float32)
    out_ref[:, 0, :] = _layer_norm_2d(t, g, b)


def _ln_phase(P, tids_col, turn_pre, g_row, b_row):
    B, sp1, H = P.shape
    bsz = 64
    grid = B // bsz
    return pl.pallas_call(
        _ln_body,
        grid=(grid,),
        in_specs=[
            pl.BlockSpec((bsz, sp1, H), lambda i: (i, 0, 0)),
            pl.BlockSpec((bsz, 1), lambda i: (i, 0)),
            pl.BlockSpec(turn_pre.shape, lambda i: (0, 0)),
            pl.BlockSpec((1, H), lambda i: (0, 0)),
            pl.BlockSpec((1, H), lambda i: (0, 0)),
        ],
        out_specs=pl.BlockSpec((bsz, sp1 + 1, H), lambda i: (i, 0, 0)),
        out_shape=jax.ShapeDtypeStruct((B, sp1 + 1, H), jnp.float32),
    )(P, tids_col, turn_pre, g_row, b_row)


def kernel(turn_ids, slot_ids, value_ids, edge_types, turn_table, slot_table,
           value_table, edge_table, W_turn, b_turn, W_bs, b_bs, W_sv, b_sv,
           ln_g, ln_b):
    B = turn_ids.shape[0]
    S = slot_ids.shape[1]
    H = turn_table.shape[1]

    i32 = jnp.int32
    tids_col = jnp.asarray(turn_ids, i32).reshape(B, 1)
    sids = jnp.asarray(slot_ids, i32)
    vids = jnp.asarray(value_ids, i32)
    etypes_col = jnp.asarray(edge_types, i32).reshape(B, 1)

    turn_pre, cat_table, edge_attr = _project(
        S, turn_table, slot_table, value_table, W_turn, W_bs, W_sv,
        b_turn.reshape(1, H), b_sv.reshape(1, H), b_bs.reshape(1, H),
        edge_table, etypes_col)

    ni = (2 * S + 7) // 8 * 8
    pad = jnp.zeros((B, ni - 2 * S), jnp.int32)
    cat_ids = jnp.concatenate([sids, vids + slot_table.shape[0], pad], axis=1)
    P = _sc_gather(cat_ids, cat_table, S, H)

    node_features = _ln_phase(P, tids_col, turn_pre,
                              ln_g.reshape(1, H), ln_b.reshape(1, H))
    return node_features, edge_attr
